# baseline (speedup 1.0000x reference)
import functools

import jax
import jax.numpy as jnp
from jax import lax
from jax.experimental import pallas as pl
from jax.experimental.pallas import tpu as pltpu
from jax.experimental.pallas import tpu_sc as plsc

EMB = 64
CHUNK = 320
NBUF = 4
AHEAD = 2
BLK = 128


@functools.partial(jax.jit, static_argnames=("B", "D", "C"))
def _gather(idx, table, B, D, C):
    info = plsc.get_sparse_core_info()
    NC, NS = info.num_cores, info.num_subcores
    NW = NC * NS
    V = table.shape[0]
    b_per_w = B // NW
    iters = b_per_w // C
    assert iters % NBUF == 0
    mesh = plsc.VectorSubcoreMesh(core_axis_name="c", subcore_axis_name="s")

    @functools.partial(
        pl.kernel,
        mesh=mesh,
        out_type=jax.ShapeDtypeStruct((B, D), jnp.float32),
        scratch_types=[
            pltpu.VMEM((iters, C), jnp.int32),
            pltpu.VMEM((NBUF, C, D), jnp.float32),
            pltpu.VMEM_SHARED((V, D), jnp.float32),
        ]
        + [pltpu.SemaphoreType.DMA] * (2 * NBUF),
        compiler_params=pltpu.CompilerParams(use_tc_tiling_on_sc=False),
    )
    def k(idx_hbm, table_hbm, out_hbm, idx_v, rows_v, tbl_sh, *sems):
        sem_g = sems[:NBUF]
        sem_s = sems[NBUF:]
        wid = lax.axis_index("s") * NC + lax.axis_index("c")
        base = wid * b_per_w

        @pl.when(lax.axis_index("s") == 0)
        def _():
            pltpu.sync_copy(table_hbm, tbl_sh)

        plsc.subcore_barrier()

        pltpu.sync_copy(idx_hbm.at[wid], idx_v)

        def gather_start(g, b):
            pltpu.async_copy(tbl_sh.at[idx_v.at[g]], rows_v.at[b], sem_g[b])

        def store_start(g, b):
            pltpu.async_copy(
                rows_v.at[b], out_hbm.at[pl.ds(base + g * C, C)], sem_s[b]
            )

        for b in range(AHEAD):
            gather_start(b, b)

        def body(i, carry):
            for b in range(NBUF):
                g = i * NBUF + b
                bn = (b + AHEAD) % NBUF

                @pl.when(g + AHEAD < iters)
                def _():
                    @pl.when(g + AHEAD >= NBUF)
                    def _():
                        pltpu.make_async_copy(
                            rows_v.at[bn],
                            out_hbm.at[pl.ds(base, C)],
                            sem_s[bn],
                        ).wait()

                    gather_start(g + AHEAD, bn)

                pltpu.make_async_copy(
                    tbl_sh.at[idx_v.at[g]], rows_v.at[b], sem_g[b]
                ).wait()
                store_start(g, b)
            return carry

        lax.fori_loop(0, iters // NBUF, body, 0)

        for g in range(iters - AHEAD, iters):
            b = g % NBUF
            pltpu.make_async_copy(
                rows_v.at[b], out_hbm.at[pl.ds(base, C)], sem_s[b]
            ).wait()

    return k(idx, table)


def _unpack_body(x_ref, o_ref):
    x = x_ref[...]
    n = x.shape[0]
    xl = x[:, :EMB]
    xr = x[:, EMB:]
    o_ref[...] = jnp.stack([xl, xr], axis=1).reshape(BLK, -1, EMB)


@functools.partial(jax.jit, static_argnames=("B0", "H", "D"))
def _unpack(x, B0, H, D):
    return pl.pallas_call(
        _unpack_body,
        grid=(B0 // BLK,),
        in_specs=[pl.BlockSpec((BLK * H // 2, 2 * D), lambda i: (i, 0))],
        out_specs=pl.BlockSpec((BLK, H, D), lambda i: (i, 0, 0)),
        out_shape=jax.ShapeDtypeStruct((B0, H, D), jnp.float32),
    )(x)


def kernel(indices, emb_table):
    B0, H = indices.shape
    B = B0 * H
    idx = indices.astype(jnp.int32).reshape(32, B // (32 * CHUNK), CHUNK)
    flat = _gather(idx, emb_table, B, EMB, CHUNK)
    return _unpack(flat.reshape(B // 2, 2 * EMB), B0, H, EMB)


# speedup vs baseline: 1.8027x; 1.8027x over previous
import functools

import jax
import jax.numpy as jnp
from jax import lax
from jax.experimental import pallas as pl
from jax.experimental.pallas import tpu as pltpu
from jax.experimental.pallas import tpu_sc as plsc

EMB = 64
NB = 2
HP = 56
NBUF = 4
AHEAD = 2


@functools.partial(jax.jit, static_argnames=("B0", "H", "D"))
def _gather(idx, table, B0, H, D):
    info = plsc.get_sparse_core_info()
    NC, NS = info.num_cores, info.num_subcores
    NW = NC * NS
    V = table.shape[0]
    b_per_w = B0 // NW
    iters = b_per_w // NB
    assert iters % NBUF == 0
    mesh = plsc.VectorSubcoreMesh(core_axis_name="c", subcore_axis_name="s")

    @functools.partial(
        pl.kernel,
        mesh=mesh,
        out_type=jax.ShapeDtypeStruct((B0, H, D), jnp.float32),
        scratch_types=[
            pltpu.VMEM((iters, NB * HP), jnp.int32),
            pltpu.VMEM((NBUF, NB, H, D), jnp.float32),
            pltpu.VMEM_SHARED((V, D), jnp.float32),
        ]
        + [pltpu.SemaphoreType.DMA] * (2 * NBUF),
        compiler_params=pltpu.CompilerParams(use_tc_tiling_on_sc=True),
    )
    def k(idx_hbm, table_hbm, out_hbm, idx_v, rows_v, tbl_sh, *sems):
        sem_g = sems[:NBUF]
        sem_s = sems[NBUF:]
        wid = lax.axis_index("s") * NC + lax.axis_index("c")
        base = wid * b_per_w

        @pl.when(lax.axis_index("s") == 0)
        def _():
            pltpu.sync_copy(table_hbm, tbl_sh)

        plsc.subcore_barrier()

        pltpu.sync_copy(idx_hbm.at[wid], idx_v)

        def gather_start(g, b):
            for j in range(NB):
                pltpu.async_copy(
                    tbl_sh.at[idx_v.at[g, pl.ds(j * HP, H)]],
                    rows_v.at[b, j],
                    sem_g[b],
                )

        def gather_wait(g, b):
            for j in range(NB):
                pltpu.make_async_copy(
                    tbl_sh.at[idx_v.at[g, pl.ds(j * HP, H)]],
                    rows_v.at[b, j],
                    sem_g[b],
                ).wait()

        def store_start(g, b):
            pltpu.async_copy(
                rows_v.at[b], out_hbm.at[pl.ds(base + g * NB, NB)], sem_s[b]
            )

        def store_wait(b):
            pltpu.make_async_copy(
                rows_v.at[b], out_hbm.at[pl.ds(base, NB)], sem_s[b]
            ).wait()

        for b in range(AHEAD):
            gather_start(b, b)

        def body(i, carry):
            for b in range(NBUF):
                g = i * NBUF + b
                bn = (b + AHEAD) % NBUF

                @pl.when(g + AHEAD < iters)
                def _():
                    @pl.when(g + AHEAD >= NBUF)
                    def _():
                        store_wait(bn)

                    gather_start(g + AHEAD, bn)

                gather_wait(g, b)
                store_start(g, b)
            return carry

        lax.fori_loop(0, iters // NBUF, body, 0)

        for g in range(iters - AHEAD, iters):
            store_wait(g % NBUF)

    return k(idx, table)


def kernel(indices, emb_table):
    B0, H = indices.shape
    idx = jnp.pad(indices.astype(jnp.int32), ((0, 0), (0, HP - H)))
    idx = idx.reshape(32, B0 // (32 * NB), NB * HP)
    return _gather(idx, emb_table, B0, H, EMB)
